# baseline (speedup 1.0000x reference)
import functools

import jax
import jax.numpy as jnp
from jax import lax
from jax.experimental import pallas as pl
from jax.experimental.pallas import tpu as pltpu
from jax.experimental.pallas import tpu_sc as plsc

VOCAB = 1000000
EMB = 64
B = 4096
L = 200
NTOK = B * L
NC = 2
NS = 16
NW = NC * NS
ROWS_PER_W = NTOK // NW
CHUNK = 512
NCHUNK = ROWS_PER_W // CHUNK

_mesh = plsc.VectorSubcoreMesh(core_axis_name="c", subcore_axis_name="s")


@functools.partial(
    pl.kernel,
    mesh=_mesh,
    out_type=jax.ShapeDtypeStruct((NTOK, EMB), jnp.float32),
    scratch_types=[
        pltpu.VMEM((CHUNK,), jnp.int32),
        pltpu.VMEM((CHUNK, EMB), jnp.float32),
        pltpu.SemaphoreType.DMA,
    ],
    compiler_params=pltpu.CompilerParams(use_tc_tiling_on_sc=False),
)
def _embed_sc(tokens_hbm, table_hbm, out_hbm, idx_v, rows_v, gsem):
    wid = lax.axis_index("s") * NC + lax.axis_index("c")
    base = wid * ROWS_PER_W

    def _chunk(j, carry):
        off = base + j * CHUNK
        pltpu.sync_copy(tokens_hbm.at[pl.ds(off, CHUNK)], idx_v)
        pltpu.async_copy(table_hbm.at[idx_v], rows_v, gsem).wait()
        pltpu.sync_copy(rows_v, out_hbm.at[pl.ds(off, CHUNK)])
        return carry

    lax.fori_loop(0, NCHUNK, _chunk, 0)


def kernel(tokens, table):
    flat = tokens.reshape(NTOK).astype(jnp.int32)
    out = _embed_sc(flat, table)
    return out.reshape(B, L, EMB)


# speedup vs baseline: 1.0406x; 1.0406x over previous
import functools

import jax
import jax.numpy as jnp
from jax import lax
from jax.experimental import pallas as pl
from jax.experimental.pallas import tpu as pltpu
from jax.experimental.pallas import tpu_sc as plsc

VOCAB = 1000000
EMB = 64
B = 4096
L = 200
NTOK = B * L
NC = 2
NS = 16
NW = NC * NS
ROWS_PER_W = NTOK // NW
CHUNK = 512
NCHUNK = ROWS_PER_W // CHUNK

_mesh = plsc.VectorSubcoreMesh(core_axis_name="c", subcore_axis_name="s")


@functools.partial(
    pl.kernel,
    mesh=_mesh,
    out_type=jax.ShapeDtypeStruct((NTOK, EMB), jnp.float32),
    scratch_types=[
        pltpu.VMEM((ROWS_PER_W,), jnp.int32),
        pltpu.VMEM((2, CHUNK, EMB), jnp.float32),
        pltpu.SemaphoreType.DMA,
        pltpu.SemaphoreType.DMA,
        pltpu.SemaphoreType.DMA,
        pltpu.SemaphoreType.DMA,
    ],
    compiler_params=pltpu.CompilerParams(use_tc_tiling_on_sc=False),
)
def _embed_sc(tokens_hbm, table_hbm, out_hbm, idx_v, rows_v, gs0, gs1, os0, os1):
    wid = lax.axis_index("s") * NC + lax.axis_index("c")
    base = wid * ROWS_PER_W
    gsems = (gs0, gs1)
    osems = (os0, os1)

    pltpu.sync_copy(tokens_hbm.at[pl.ds(base, ROWS_PER_W)], idx_v)

    def gather(j, s):
        pltpu.async_copy(
            table_hbm.at[idx_v.at[pl.ds(j * CHUNK, CHUNK)]], rows_v.at[s], gsems[s]
        )

    def put(j, s):
        pltpu.async_copy(
            rows_v.at[s], out_hbm.at[pl.ds(base + j * CHUNK, CHUNK)], osems[s]
        )

    def wait_gather(s):
        pltpu.make_async_copy(
            table_hbm.at[pl.ds(0, CHUNK)], rows_v.at[s], gsems[s]
        ).wait()

    def wait_put(s):
        pltpu.make_async_copy(
            rows_v.at[s], out_hbm.at[pl.ds(base, CHUNK)], osems[s]
        ).wait()

    gather(0, 0)

    def round_body(r, carry):
        j0 = r * 2
        for s in range(2):
            j = j0 + s
            wait_gather(s)

            @pl.when(j >= 1)
            def _():
                wait_put(1 - s)

            @pl.when(j + 1 < NCHUNK)
            def _():
                gather(j + 1, 1 - s)

            put(j, s)
        return carry

    lax.fori_loop(0, NCHUNK // 2, round_body, 0)
    wait_put(1)


def kernel(tokens, table):
    flat = tokens.reshape(NTOK).astype(jnp.int32)
    out = _embed_sc(flat, table)
    return out.reshape(B, L, EMB)


# speedup vs baseline: 1.0437x; 1.0029x over previous
import functools

import jax
import jax.numpy as jnp
from jax import lax
from jax.experimental import pallas as pl
from jax.experimental.pallas import tpu as pltpu
from jax.experimental.pallas import tpu_sc as plsc

VOCAB = 1000000
EMB = 64
B = 4096
L = 200
NTOK = B * L
NC = 2
NS = 16
NW = NC * NS
ROWS_PER_W = NTOK // NW
CHUNK = 512
NCHUNK = ROWS_PER_W // CHUNK

_mesh = plsc.VectorSubcoreMesh(core_axis_name="c", subcore_axis_name="s")


@functools.partial(
    pl.kernel,
    mesh=_mesh,
    out_type=jax.ShapeDtypeStruct((NTOK, EMB), jnp.float32),
    scratch_types=[
        pltpu.VMEM((ROWS_PER_W,), jnp.int32),
        pltpu.VMEM((2, CHUNK, EMB), jnp.float32),
        pltpu.SemaphoreType.DMA,
        pltpu.SemaphoreType.DMA,
        pltpu.SemaphoreType.DMA,
        pltpu.SemaphoreType.DMA,
    ],
    compiler_params=pltpu.CompilerParams(
        use_tc_tiling_on_sc=False, skip_device_barrier=True
    ),
)
def _embed_sc(tokens_hbm, table_hbm, out_hbm, idx_v, rows_v, gs0, gs1, os0, os1):
    wid = lax.axis_index("s") * NC + lax.axis_index("c")
    base = wid * ROWS_PER_W
    gsems = (gs0, gs1)
    osems = (os0, os1)

    pltpu.sync_copy(tokens_hbm.at[pl.ds(base, ROWS_PER_W)], idx_v)

    def gather(j, s):
        pltpu.async_copy(
            table_hbm.at[idx_v.at[pl.ds(j * CHUNK, CHUNK)]], rows_v.at[s], gsems[s]
        )

    def put(j, s):
        pltpu.async_copy(
            rows_v.at[s], out_hbm.at[pl.ds(base + j * CHUNK, CHUNK)], osems[s]
        )

    def wait_gather(s):
        pltpu.make_async_copy(
            table_hbm.at[pl.ds(0, CHUNK)], rows_v.at[s], gsems[s]
        ).wait()

    def wait_put(s):
        pltpu.make_async_copy(
            rows_v.at[s], out_hbm.at[pl.ds(base, CHUNK)], osems[s]
        ).wait()

    gather(0, 0)

    def round_body(r, carry):
        j0 = r * 2
        for s in range(2):
            j = j0 + s
            wait_gather(s)

            @pl.when(j >= 1)
            def _():
                wait_put(1 - s)

            @pl.when(j + 1 < NCHUNK)
            def _():
                gather(j + 1, 1 - s)

            put(j, s)
        return carry

    lax.fori_loop(0, NCHUNK // 2, round_body, 0)
    wait_put(1)


def kernel(tokens, table):
    flat = tokens.reshape(NTOK).astype(jnp.int32)
    out = _embed_sc(flat, table)
    return out.reshape(B, L, EMB)
